# baseline (speedup 1.0000x reference)
import itertools

import jax
import jax.numpy as jnp
from jax.experimental import pallas as pl

_RESO_3D = [32, 64, 128]
_OFF_3D = [0, 32768, 294912, 2392064]
_RESO_2D = [128, 256, 512]
_OFF_2D = [0, 16384, 81920, 344064]


def _cf(xyz, R):
    pos = jnp.clip(xyz, 0.0, 1.0) * (R - 1)
    cell = jnp.clip(jnp.floor(pos), 0.0, R - 2).astype(jnp.int32)
    return cell, pos - cell.astype(pos.dtype)


def _flat(cell, corner, R):
    D = cell.shape[1]
    mult = jnp.asarray([R ** (D - 1 - d) for d in range(D)], dtype=jnp.int32)
    return jnp.sum((cell + jnp.asarray(corner, dtype=jnp.int32)) * mult, axis=1)


def _cw(frac, corner):
    c = jnp.asarray(corner)
    return jnp.prod(jnp.where(c == 1, frac, 1.0 - frac), axis=1)


def _creater(xyz, feature, reso, off):
    D = xyz.shape[1]
    grids = []
    for l, R in enumerate(reso):
        cell, frac = _cf(xyz, R)
        size = off[l + 1] - off[l]
        acc = jnp.zeros((size, feature.shape[1]), dtype=feature.dtype)
        cnt = jnp.zeros((size,), dtype=feature.dtype)
        for corner in itertools.product((0, 1), repeat=D):
            w = _cw(frac, corner)
            flat = _flat(cell, corner, R)
            acc = acc.at[flat].add(w[:, None] * feature)
            cnt = cnt.at[flat].add(w)
        grids.append(acc / jnp.maximum(cnt, 1e-8)[:, None])
    return jnp.concatenate(grids, axis=0)


def _encoder(xyz, grid, off, reso):
    D = xyz.shape[1]
    outs = []
    for l, R in enumerate(reso):
        cell, frac = _cf(xyz, R)
        out = jnp.zeros((xyz.shape[0], grid.shape[1]), dtype=grid.dtype)
        for corner in itertools.product((0, 1), repeat=D):
            w = _cw(frac, corner)
            flat = _flat(cell, corner, R)
            out = out + w[:, None] * grid[off[l] + flat]
        outs.append(out)
    return jnp.concatenate(outs, axis=-1)


def _copy_body(x_ref, o_ref):
    o_ref[...] = x_ref[...]


def kernel(xyz_for_creater, xyz_for_interp, feature):
    g3 = _creater(xyz_for_creater, feature, _RESO_3D, _OFF_3D)
    gxy = _creater(xyz_for_creater[:, 0:2], feature, _RESO_2D, _OFF_2D)
    gxz = _creater(xyz_for_creater[:, 0::2], feature, _RESO_2D, _OFF_2D)
    gyz = _creater(xyz_for_creater[:, 1:3], feature, _RESO_2D, _OFF_2D)
    c3 = _encoder(xyz_for_interp, g3, _OFF_3D, _RESO_3D)
    cxy = _encoder(xyz_for_interp[:, 0:2], gxy, _OFF_2D, _RESO_2D)
    cxz = _encoder(xyz_for_interp[:, 0::2], gxz, _OFF_2D, _RESO_2D)
    cyz = _encoder(xyz_for_interp[:, 1:3], gyz, _OFF_2D, _RESO_2D)
    out = jnp.concatenate([c3, cxy, cxz, cyz], axis=-1)
    return pl.pallas_call(
        _copy_body,
        grid=(100,),
        in_specs=[pl.BlockSpec((2000, 48), lambda i: (i, 0))],
        out_specs=pl.BlockSpec((2000, 48), lambda i: (i, 0)),
        out_shape=jax.ShapeDtypeStruct(out.shape, out.dtype),
    )(out)


# speedup vs baseline: 17.9048x; 17.9048x over previous
import functools
import itertools

import jax
import jax.numpy as jnp
from jax import lax
from jax.experimental import pallas as pl
from jax.experimental.pallas import tpu as pltpu
from jax.experimental.pallas import tpu_sc as plsc

N = 200000
NPAD = 204800
PP = NPAD // 32
NV = PP // 16
F32 = jnp.float32
I32 = jnp.int32

CAP = 512
FJ = CAP // 128

JOBS = [
    (3, 32, 1, 32768, 0, (0, 1, 2)),
    (3, 64, 2, 131072, 32768, (0, 1, 2)),
    (2, 128, 1, 16384, 294912, (0, 1)),
    (2, 256, 1, 65536, 311296, (0, 1)),
    (2, 512, 2, 131072, 376832, (0, 1)),
    (2, 128, 1, 16384, 638976, (0, 2)),
    (2, 256, 1, 65536, 655360, (0, 2)),
    (2, 512, 2, 131072, 720896, (0, 2)),
    (2, 128, 1, 16384, 983040, (1, 2)),
    (2, 256, 1, 65536, 999424, (1, 2)),
    (2, 512, 2, 131072, 1064960, (1, 2)),
    (3, 128, 15, 147456, 1327104, (0, 1, 2)),
]
TOT = 3424256
TOTP = 3538944
CHS = 147456 + 128

LEVELS = [
    (3, 32, 0, 0, (0, 1, 2)),
    (3, 64, 32768, 4, (0, 1, 2)),
    (3, 128, 1327104, 8, (0, 1, 2)),
    (2, 128, 294912, 12, (0, 1)),
    (2, 256, 311296, 16, (0, 1)),
    (2, 512, 376832, 20, (0, 1)),
    (2, 128, 638976, 24, (0, 2)),
    (2, 256, 655360, 28, (0, 2)),
    (2, 512, 720896, 32, (0, 2)),
    (2, 128, 983040, 36, (1, 2)),
    (2, 256, 999424, 40, (1, 2)),
    (2, 512, 1064960, 44, (1, 2)),
]
PB = 320
NBLK = PP // PB

_CP = pltpu.CompilerParams(needs_layout_passes=False, use_tc_tiling_on_sc=False)


def _iota():
    return lax.iota(I32, 16)


def _cells(coord, R):
    pos = jnp.clip(coord, 0.0, 1.0) * (R - 1)
    cell = jnp.minimum(pos.astype(I32), R - 2)
    frac = pos - cell.astype(F32)
    return cell, frac


def _k1_body(pts, fts, p_out, chunk, xb, fb, zbuf, stage, idxb, fsem, zsem):
    cid = lax.axis_index("c")
    sid = lax.axis_index("s")
    wid = sid * 2 + cid
    iota = _iota()
    gbase = wid * PP

    for d in range(3):
        pltpu.sync_copy(pts.at[d, pl.ds(gbase, PP)], xb.at[d])
    for d in range(4):
        pltpu.sync_copy(fts.at[d, pl.ds(gbase, PP)], fb.at[d])

    def _z(r, carry):
        plsc.store_scatter(zbuf, [r * 2 + (iota >> 3), iota & 7],
                           jnp.zeros((16,), F32))
        return carry
    lax.fori_loop(0, 64, _z, 0)

    def drain_fj():
        for j in range(FJ):
            pltpu.make_async_copy(stage.at[0, pl.ds(j * 128, 128), :],
                                  chunk.at[idxb.at[0, j]], fsem).wait()

    for dims, R, nch, csz, pbase, dd in JOBS:
        pl_major = csz // (R * R) if dims == 3 else csz // R
        zq = csz // 16
        cs16 = csz // 16
        corners = list(itertools.product((0, 1), repeat=dims))

        def one_chunk(c, dims=dims, R=R, nch=nch, csz=csz, pbase=pbase,
                      dd=dd, pl_major=pl_major, zq=zq, cs16=cs16,
                      corners=corners):
            for k in range(zq // 128):
                pltpu.async_copy(zbuf, chunk.at[pl.ds(sid * zq + k * 128, 128), :], zsem)
            for k in range(zq // 128):
                pltpu.make_async_copy(zbuf, chunk.at[pl.ds(sid * zq + k * 128, 128), :], zsem).wait()
            plsc.subcore_barrier()

            cb = c * csz

            def vec(v, carry):
                ptr, b, fc = carry
                base = v * 16
                valid = (gbase + base + iota) < N
                cs_ = []
                fr_ = []
                for d in dd:
                    cell, frac = _cells(xb[d, pl.ds(base, 16)], R)
                    cs_.append(cell)
                    fr_.append(frac)
                fv = [fb[d, pl.ds(base, 16)] for d in range(4)]
                ib = jnp.full((16,), b, I32)
                for corner in corners:
                    w = jnp.ones((16,), F32)
                    for d in range(dims):
                        w = w * (fr_[d] if corner[d] == 1 else 1.0 - fr_[d])
                    cc0 = cs_[0] + corner[0]
                    flat = cc0
                    for d in range(1, dims):
                        flat = flat * R + cs_[d] + corner[d]
                    m = valid
                    if nch > 1:
                        lo = c * pl_major
                        m = m & (cc0 >= lo) & (cc0 < lo + pl_major)
                    lrow = flat - cb
                    csum = plsc.cumsum(m.astype(I32))
                    pc = jnp.max(csum)
                    posv = ptr + csum - 1
                    vals = (w * fv[0], w * fv[1], w * fv[2], w * fv[3], w)
                    for ci, val in enumerate(vals):
                        plsc.store_scatter(stage, [ib, posv, jnp.full((16,), ci, I32)],
                                           val, mask=m)
                    plsc.store_scatter(idxb, [ib, posv >> 7, posv & 127], lrow, mask=m)
                    ptr = ptr + pc

                def fire(a):
                    ptr, b, fc = a
                    ibf = jnp.full((16,), b, I32)
                    for jj in range(8):
                        p = ptr + jj * 16 + iota
                        plsc.store_scatter(idxb, [ibf, p >> 7, p & 127],
                                           csz + (p & 127), mask=p < CAP)
                    for j in range(FJ):
                        pltpu.async_copy(stage.at[b, pl.ds(j * 128, 128), :],
                                         chunk.at[idxb.at[b, j]], fsem, add=True)

                    @pl.when(fc >= 1)
                    def _():
                        drain_fj()
                    return jnp.int32(0), 1 - b, fc + 1

                return lax.cond(ptr >= CAP - 128, fire, lambda a: a, (ptr, b, fc))

            ptr, b, fc = lax.fori_loop(
                0, NV, vec, (jnp.int32(0), jnp.int32(0), jnp.int32(0)))

            ibf = jnp.full((16,), b, I32)
            for j in range(FJ):
                for jj in range(8):
                    p = j * 128 + jj * 16 + iota
                    plsc.store_scatter(idxb, [ibf, jnp.full((16,), j, I32),
                                              jj * 16 + iota],
                                       csz + (p & 127), mask=p >= ptr)

            for j in range(FJ):
                pltpu.async_copy(stage.at[b, pl.ds(j * 128, 128), :],
                                 chunk.at[idxb.at[b, j]], fsem, add=True)

            @pl.when(fc >= 1)
            def _():
                drain_fj()
            drain_fj()

            plsc.subcore_barrier()
            pltpu.sync_copy(chunk.at[pl.ds(sid * cs16, cs16), :],
                            p_out.at[cid, pl.ds(pbase + cb + sid * cs16, cs16), :])
            plsc.subcore_barrier()
            return 0

        if nch == 1:
            one_chunk(0)
        else:
            lax.fori_loop(0, nch, lambda c, _, f=one_chunk: (f(c), 0)[1], 0)


def _k2_body(p_in, g_out, pa, pb, og):
    cid = lax.axis_index("c")
    sid = lax.axis_index("s")
    wid = sid * 2 + cid
    iota = _iota()
    rt = TOT // 32
    rb_base = wid * rt

    def do_block(off, bn):
        pltpu.sync_copy(p_in.at[0, pl.ds(off, bn), :], pa.at[pl.ds(0, bn), :])
        pltpu.sync_copy(p_in.at[1, pl.ds(off, bn), :], pb.at[pl.ds(0, bn), :])

        def vec(vi, carry):
            r = vi * 16 + iota
            a = [plsc.load_gather(pa, [r, jnp.full((16,), ci, I32)]) for ci in range(5)]
            b = [plsc.load_gather(pb, [r, jnp.full((16,), ci, I32)]) for ci in range(5)]
            den = jnp.maximum(a[4] + b[4], 1e-8)
            for ci in range(4):
                plsc.store_scatter(og, [r, jnp.full((16,), ci, I32)],
                                   (a[ci] + b[ci]) / den)
            return carry
        lax.fori_loop(0, bn // 16, vec, 0)
        pltpu.sync_copy(og.at[pl.ds(0, bn), :], g_out.at[pl.ds(off, bn), :])

    def blk(i, carry):
        do_block(rb_base + i * 2048, 2048)
        return carry
    lax.fori_loop(0, rt // 2048, blk, 0)
    if rt % 2048:
        do_block(rb_base + (rt // 2048) * 2048, rt % 2048)


def _k3_body(qts, g_in, out, xq, wb, ibuf, rbuf, ob, gsem):
    cid = lax.axis_index("c")
    sid = lax.axis_index("s")
    wid = sid * 2 + cid
    iota = _iota()
    gbase = wid * PP
    for d in range(3):
        pltpu.sync_copy(qts.at[d, pl.ds(gbase, PP)], xq.at[d])

    segs = [(0, 128), (128, 128), (256, 64)]

    def blk(bi, carry):
        pbase = bi * PB
        handles = {}

        def prep(li):
            dims, R, goff, coff, dd = LEVELS[li]
            q = li % 2
            corners = list(itertools.product((0, 1), repeat=dims))

            def vec(v, c2):
                base = pbase + v * 16
                cs_, fr_ = [], []
                for d in dd:
                    cell, frac = _cells(xq[d, pl.ds(base, 16)], R)
                    cs_.append(cell)
                    fr_.append(frac)
                for ci, corner in enumerate(corners):
                    w = jnp.ones((16,), F32)
                    for d in range(dims):
                        w = w * (fr_[d] if corner[d] == 1 else 1.0 - fr_[d])
                    flat = cs_[0] + corner[0]
                    for d in range(1, dims):
                        flat = flat * R + cs_[d] + corner[d]
                    wb[q, ci, pl.ds(v * 16, 16)] = w
                    ibuf[q, ci, pl.ds(v * 16, 16)] = flat + goff
                return c2
            lax.fori_loop(0, PB // 16, vec, 0)
            hs = []
            for ci in range(len(corners)):
                for (o, ln) in segs:
                    hs.append(pltpu.async_copy(
                        g_in.at[ibuf.at[q, ci, pl.ds(o, ln)]],
                        rbuf.at[q, ci, pl.ds(o, ln), :], gsem.at[q]))
            handles[li] = hs

        def accum(li):
            dims, R, goff, coff, dd = LEVELS[li]
            q = li % 2
            ncor = 2 ** dims
            for h in handles[li]:
                h.wait()

            def grp(u, c2):
                r4 = u * 4 + (iota >> 2)
                colv = coff + (iota & 3)
                for ci in range(ncor):
                    rv = plsc.load_gather(rbuf.at[q, ci], [r4, iota & 3])
                    wv = plsc.load_gather(wb.at[q, ci], [r4])
                    val = rv * wv
                    if ci == 0:
                        plsc.store_scatter(ob, [r4, colv], val)
                    else:
                        plsc.addupdate_scatter(ob, [r4, colv], val)
                return c2
            lax.fori_loop(0, PB // 4, grp, 0)

        prep(0)
        for li in range(len(LEVELS)):
            if li + 1 < len(LEVELS):
                prep(li + 1)
            accum(li)
        pltpu.sync_copy(ob, out.at[pl.ds(gbase + pbase, PB), :])
        return carry

    lax.fori_loop(0, NBLK, blk, 0)


def kernel(xyz_for_creater, xyz_for_interp, feature):
    mesh = plsc.VectorSubcoreMesh(core_axis_name="c", subcore_axis_name="s")

    pts = jnp.pad(xyz_for_creater, ((0, NPAD - N), (0, 0))).T
    fts = jnp.pad(feature, ((0, NPAD - N), (0, 0))).T
    qts = jnp.pad(xyz_for_interp, ((0, NPAD - N), (0, 0))).T

    k1 = pl.kernel(
        _k1_body,
        out_type=jax.ShapeDtypeStruct((2, TOTP, 8), F32),
        mesh=mesh,
        compiler_params=_CP,
        scratch_types=[
            pltpu.VMEM_SHARED((CHS, 8), F32),
            pltpu.VMEM((3, PP), F32),
            pltpu.VMEM((4, PP), F32),
            pltpu.VMEM((128, 8), F32),
            pltpu.VMEM((2, CAP, 8), F32),
            pltpu.VMEM((2, FJ, 128), I32),
            pltpu.SemaphoreType.DMA,
            pltpu.SemaphoreType.DMA,
        ],
    )
    p_part = k1(pts, fts)

    k2 = pl.kernel(
        _k2_body,
        out_type=jax.ShapeDtypeStruct((TOT, 8), F32),
        mesh=mesh,
        compiler_params=_CP,
        scratch_types=[
            pltpu.VMEM((2048, 8), F32),
            pltpu.VMEM((2048, 8), F32),
            pltpu.VMEM((2048, 8), F32),
        ],
    )
    g = k2(p_part)

    k3 = pl.kernel(
        _k3_body,
        out_type=jax.ShapeDtypeStruct((NPAD, 48), F32),
        mesh=mesh,
        compiler_params=_CP,
        scratch_types=[
            pltpu.VMEM((3, PP), F32),
            pltpu.VMEM((2, 8, PB), F32),
            pltpu.VMEM((2, 8, PB), I32),
            pltpu.VMEM((2, 8, PB, 8), F32),
            pltpu.VMEM((PB, 48), F32),
            pltpu.SemaphoreType.DMA((2,)),
        ],
    )
    out = k3(qts, g)
    return out[:N]


# speedup vs baseline: 18.1231x; 1.0122x over previous
import functools
import itertools

import jax
import jax.numpy as jnp
from jax import lax
from jax.experimental import pallas as pl
from jax.experimental.pallas import tpu as pltpu
from jax.experimental.pallas import tpu_sc as plsc

N = 200000
NPAD = 204800
PP = NPAD // 32
NV = PP // 16
F32 = jnp.float32
I32 = jnp.int32

CAP = 512
FJ = CAP // 128

D3 = (0, 1, 2)
JOBS = [
    ([(3, 32, 0, D3), (2, 128, 32768, (0, 1)), (2, 128, 49152, (0, 2)),
      (2, 128, 65536, (1, 2)), (2, 256, 81920, (1, 2))], 1, 147456, 0),
    ([(3, 64, 0, D3)], 2, 131072, 147456),
    ([(2, 256, 0, (0, 1)), (2, 256, 65536, (0, 2))], 1, 131072, 409600),
    ([(2, 512, 0, (0, 1))], 2, 131072, 540672),
    ([(2, 512, 0, (0, 2))], 2, 131072, 802816),
    ([(2, 512, 0, (1, 2))], 2, 131072, 1064960),
    ([(3, 128, 0, D3)], 15, 147456, 1327104),
]
TOT = 3424256
TOTP = 3538944
CHS = 147456 + 128

LEVELS = [
    (3, 32, 0, 0, (0, 1, 2)),
    (3, 64, 147456, 4, (0, 1, 2)),
    (3, 128, 1327104, 8, (0, 1, 2)),
    (2, 128, 32768, 12, (0, 1)),
    (2, 256, 409600, 16, (0, 1)),
    (2, 512, 540672, 20, (0, 1)),
    (2, 128, 49152, 24, (0, 2)),
    (2, 256, 475136, 28, (0, 2)),
    (2, 512, 802816, 32, (0, 2)),
    (2, 128, 65536, 36, (1, 2)),
    (2, 256, 81920, 40, (1, 2)),
    (2, 512, 1064960, 44, (1, 2)),
]
PB = 320
NBLK = PP // PB

_CP = pltpu.CompilerParams(needs_layout_passes=False, use_tc_tiling_on_sc=False)


def _iota():
    return lax.iota(I32, 16)


def _cells(coord, R):
    pos = jnp.clip(coord, 0.0, 1.0) * (R - 1)
    cell = jnp.minimum(pos.astype(I32), R - 2)
    frac = pos - cell.astype(F32)
    return cell, frac


def _k1_body(pts, fts, p_out, chunk, xb, fb, zbuf, stage, idxb, fsem, zsem):
    cid = lax.axis_index("c")
    sid = lax.axis_index("s")
    wid = sid * 2 + cid
    iota = _iota()
    gbase = wid * PP

    for d in range(3):
        pltpu.sync_copy(pts.at[d, pl.ds(gbase, PP)], xb.at[d])
    for d in range(4):
        pltpu.sync_copy(fts.at[d, pl.ds(gbase, PP)], fb.at[d])

    def _z(r, carry):
        plsc.store_scatter(zbuf, [r * 2 + (iota >> 3), iota & 7],
                           jnp.zeros((16,), F32))
        return carry
    lax.fori_loop(0, 64, _z, 0)

    def drain_fj():
        for j in range(FJ):
            pltpu.make_async_copy(stage.at[0, pl.ds(j * 128, 128), :],
                                  chunk.at[idxb.at[0, j]], fsem).wait()

    for levels, nch, csz, pbase in JOBS:
        zq = csz // 16
        cs16 = csz // 16

        def one_chunk(c, levels=levels, nch=nch, csz=csz, pbase=pbase,
                      zq=zq, cs16=cs16):
            for k in range(zq // 128):
                pltpu.async_copy(zbuf, chunk.at[pl.ds(sid * zq + k * 128, 128), :], zsem)
            for k in range(zq // 128):
                pltpu.make_async_copy(zbuf, chunk.at[pl.ds(sid * zq + k * 128, 128), :], zsem).wait()
            plsc.subcore_barrier()

            cb = c * csz

            def vec(v, carry):
                ptr, b, fc = carry
                base = v * 16
                valid = (gbase + base + iota) < N
                fv = [fb[d, pl.ds(base, 16)] for d in range(4)]
                ib = jnp.full((16,), b, I32)
                for dims, R, loff, dd in levels:
                    pl_major = csz // (R * R) if dims == 3 else csz // R
                    cs_ = []
                    fr_ = []
                    for d in dd:
                        cell, frac = _cells(xb[d, pl.ds(base, 16)], R)
                        cs_.append(cell)
                        fr_.append(frac)
                    for corner in itertools.product((0, 1), repeat=dims):
                        w = jnp.ones((16,), F32)
                        for d in range(dims):
                            w = w * (fr_[d] if corner[d] == 1 else 1.0 - fr_[d])
                        cc0 = cs_[0] + corner[0]
                        flat = cc0
                        for d in range(1, dims):
                            flat = flat * R + cs_[d] + corner[d]
                        m = valid
                        if nch > 1:
                            lo = c * pl_major
                            m = m & (cc0 >= lo) & (cc0 < lo + pl_major)
                        lrow = flat - cb + loff
                        csum = plsc.cumsum(m.astype(I32))
                        pc = jnp.max(csum)
                        posv = ptr + csum - 1
                        vals = (w * fv[0], w * fv[1], w * fv[2], w * fv[3], w)
                        for ci, val in enumerate(vals):
                            plsc.store_scatter(stage, [ib, posv, jnp.full((16,), ci, I32)],
                                               val, mask=m)
                        plsc.store_scatter(idxb, [ib, posv >> 7, posv & 127], lrow, mask=m)
                        ptr = ptr + pc
                    ptr, b, fc = lax.cond(ptr >= CAP - 128, fire_fn, lambda a: a,
                                          (ptr, b, fc))
                return ptr, b, fc

            def fire_fn(a):
                ptr, b, fc = a
                ibf = jnp.full((16,), b, I32)
                for jj in range(8):
                    p = ptr + jj * 16 + iota
                    plsc.store_scatter(idxb, [ibf, p >> 7, p & 127],
                                       csz + (p & 127), mask=p < CAP)
                for j in range(FJ):
                    pltpu.async_copy(stage.at[b, pl.ds(j * 128, 128), :],
                                     chunk.at[idxb.at[b, j]], fsem, add=True)

                @pl.when(fc >= 1)
                def _():
                    drain_fj()
                return jnp.int32(0), 1 - b, fc + 1

            ptr, b, fc = lax.fori_loop(
                0, NV, vec, (jnp.int32(0), jnp.int32(0), jnp.int32(0)))

            ibf = jnp.full((16,), b, I32)
            for j in range(FJ):
                for jj in range(8):
                    p = j * 128 + jj * 16 + iota
                    plsc.store_scatter(idxb, [ibf, jnp.full((16,), j, I32),
                                              jj * 16 + iota],
                                       csz + (p & 127), mask=p >= ptr)

            for j in range(FJ):
                pltpu.async_copy(stage.at[b, pl.ds(j * 128, 128), :],
                                 chunk.at[idxb.at[b, j]], fsem, add=True)

            @pl.when(fc >= 1)
            def _():
                drain_fj()
            drain_fj()

            plsc.subcore_barrier()
            pltpu.sync_copy(chunk.at[pl.ds(sid * cs16, cs16), :],
                            p_out.at[cid, pl.ds(pbase + cb + sid * cs16, cs16), :])
            plsc.subcore_barrier()
            return 0

        if nch == 1:
            one_chunk(0)
        else:
            lax.fori_loop(0, nch, lambda c, _, f=one_chunk: (f(c), 0)[1], 0)


def _k2_body(p_in, g_out, pa, pb, og, rsem):
    cid = lax.axis_index("c")
    sid = lax.axis_index("s")
    wid = sid * 2 + cid
    iota = _iota()
    rt = TOT // 32
    rb_base = wid * rt

    def do_block(off, bn):
        h0 = pltpu.async_copy(p_in.at[0, pl.ds(off, bn), :], pa.at[pl.ds(0, bn), :], rsem)
        h1 = pltpu.async_copy(p_in.at[1, pl.ds(off, bn), :], pb.at[pl.ds(0, bn), :], rsem)
        h0.wait()
        h1.wait()

        def vec(vi, carry):
            r = vi * 16 + iota
            a = [plsc.load_gather(pa, [r, jnp.full((16,), ci, I32)]) for ci in range(5)]
            b = [plsc.load_gather(pb, [r, jnp.full((16,), ci, I32)]) for ci in range(5)]
            den = jnp.maximum(a[4] + b[4], 1e-8)
            for ci in range(4):
                plsc.store_scatter(og, [r, jnp.full((16,), ci, I32)],
                                   (a[ci] + b[ci]) / den)
            return carry
        lax.fori_loop(0, bn // 16, vec, 0)
        pltpu.sync_copy(og.at[pl.ds(0, bn), :], g_out.at[pl.ds(off, bn), :])

    def blk(i, carry):
        do_block(rb_base + i * 2048, 2048)
        return carry
    lax.fori_loop(0, rt // 2048, blk, 0)
    if rt % 2048:
        do_block(rb_base + (rt // 2048) * 2048, rt % 2048)


def _k3_body(qts, g_in, out, xq, wb, ibuf, rbuf, ob, gsem):
    cid = lax.axis_index("c")
    sid = lax.axis_index("s")
    wid = sid * 2 + cid
    iota = _iota()
    gbase = wid * PP
    for d in range(3):
        pltpu.sync_copy(qts.at[d, pl.ds(gbase, PP)], xq.at[d])

    segs = [(0, 128), (128, 128), (256, 64)]

    def blk(bi, carry):
        pbase = bi * PB
        handles = {}

        def prep(li):
            dims, R, goff, coff, dd = LEVELS[li]
            q = li % 2
            corners = list(itertools.product((0, 1), repeat=dims))

            def vec(v, c2):
                base = pbase + v * 16
                cs_, fr_ = [], []
                for d in dd:
                    cell, frac = _cells(xq[d, pl.ds(base, 16)], R)
                    cs_.append(cell)
                    fr_.append(frac)
                for ci, corner in enumerate(corners):
                    w = jnp.ones((16,), F32)
                    for d in range(dims):
                        w = w * (fr_[d] if corner[d] == 1 else 1.0 - fr_[d])
                    flat = cs_[0] + corner[0]
                    for d in range(1, dims):
                        flat = flat * R + cs_[d] + corner[d]
                    wb[q, ci, pl.ds(v * 16, 16)] = w
                    ibuf[q, ci, pl.ds(v * 16, 16)] = flat + goff
                return c2
            lax.fori_loop(0, PB // 16, vec, 0)
            hs = []
            for ci in range(len(corners)):
                for (o, ln) in segs:
                    hs.append(pltpu.async_copy(
                        g_in.at[ibuf.at[q, ci, pl.ds(o, ln)]],
                        rbuf.at[q, ci, pl.ds(o, ln), :], gsem.at[q]))
            handles[li] = hs

        def accum(li):
            dims, R, goff, coff, dd = LEVELS[li]
            q = li % 2
            ncor = 2 ** dims
            for h in handles[li]:
                h.wait()

            def grp(u, c2):
                r4 = u * 4 + (iota >> 2)
                colv = coff + (iota & 3)
                for ci in range(ncor):
                    rv = plsc.load_gather(rbuf.at[q, ci], [r4, iota & 3])
                    wv = plsc.load_gather(wb.at[q, ci], [r4])
                    val = rv * wv
                    if ci == 0:
                        plsc.store_scatter(ob, [r4, colv], val)
                    else:
                        plsc.addupdate_scatter(ob, [r4, colv], val)
                return c2
            lax.fori_loop(0, PB // 4, grp, 0)

        prep(0)
        for li in range(len(LEVELS)):
            if li + 1 < len(LEVELS):
                prep(li + 1)
            accum(li)
        pltpu.sync_copy(ob, out.at[pl.ds(gbase + pbase, PB), :])
        return carry

    lax.fori_loop(0, NBLK, blk, 0)


def kernel(xyz_for_creater, xyz_for_interp, feature):
    mesh = plsc.VectorSubcoreMesh(core_axis_name="c", subcore_axis_name="s")

    pts = jnp.pad(xyz_for_creater, ((0, NPAD - N), (0, 0))).T
    fts = jnp.pad(feature, ((0, NPAD - N), (0, 0))).T
    qts = jnp.pad(xyz_for_interp, ((0, NPAD - N), (0, 0))).T

    k1 = pl.kernel(
        _k1_body,
        out_type=jax.ShapeDtypeStruct((2, TOTP, 8), F32),
        mesh=mesh,
        compiler_params=_CP,
        scratch_types=[
            pltpu.VMEM_SHARED((CHS, 8), F32),
            pltpu.VMEM((3, PP), F32),
            pltpu.VMEM((4, PP), F32),
            pltpu.VMEM((128, 8), F32),
            pltpu.VMEM((2, CAP, 8), F32),
            pltpu.VMEM((2, FJ, 128), I32),
            pltpu.SemaphoreType.DMA,
            pltpu.SemaphoreType.DMA,
        ],
    )
    p_part = k1(pts, fts)

    k2 = pl.kernel(
        _k2_body,
        out_type=jax.ShapeDtypeStruct((TOT, 8), F32),
        mesh=mesh,
        compiler_params=_CP,
        scratch_types=[
            pltpu.VMEM((2048, 8), F32),
            pltpu.VMEM((2048, 8), F32),
            pltpu.VMEM((2048, 8), F32),
            pltpu.SemaphoreType.DMA,
        ],
    )
    g = k2(p_part)

    k3 = pl.kernel(
        _k3_body,
        out_type=jax.ShapeDtypeStruct((NPAD, 48), F32),
        mesh=mesh,
        compiler_params=_CP,
        scratch_types=[
            pltpu.VMEM((3, PP), F32),
            pltpu.VMEM((2, 8, PB), F32),
            pltpu.VMEM((2, 8, PB), I32),
            pltpu.VMEM((2, 8, PB, 8), F32),
            pltpu.VMEM((PB, 48), F32),
            pltpu.SemaphoreType.DMA((2,)),
        ],
    )
    out = k3(qts, g)
    return out[:N]


# speedup vs baseline: 18.2285x; 1.0058x over previous
import functools
import itertools

import jax
import jax.numpy as jnp
from jax import lax
from jax.experimental import pallas as pl
from jax.experimental.pallas import tpu as pltpu
from jax.experimental.pallas import tpu_sc as plsc

N = 200000
NPAD = 204800
PP = NPAD // 32
NV = PP // 16
F32 = jnp.float32
I32 = jnp.int32

CAP = 512
FJ = CAP // 128

D3 = (0, 1, 2)
JOBS = [
    ([(3, 32, 0, D3), (2, 128, 32768, (0, 1)), (2, 128, 49152, (0, 2)),
      (2, 128, 65536, (1, 2)), (2, 256, 81920, (1, 2))], 1, 147456, 0),
    ([(3, 64, 0, D3)], 2, 131072, 147456),
    ([(2, 256, 0, (0, 1)), (2, 256, 65536, (0, 2))], 1, 131072, 409600),
    ([(2, 512, 0, (0, 1))], 2, 131072, 540672),
    ([(2, 512, 0, (0, 2))], 2, 131072, 802816),
    ([(2, 512, 0, (1, 2))], 2, 131072, 1064960),
    ([(3, 128, 0, D3)], 15, 147456, 1327104),
]
TOT = 3424256
TOTP = 3538944
CHS = 147456 + 128

LEVELS = [
    (3, 32, 0, 0, (0, 1, 2)),
    (3, 64, 147456, 4, (0, 1, 2)),
    (3, 128, 1327104, 8, (0, 1, 2)),
    (2, 128, 32768, 12, (0, 1)),
    (2, 256, 409600, 16, (0, 1)),
    (2, 512, 540672, 20, (0, 1)),
    (2, 128, 49152, 24, (0, 2)),
    (2, 256, 475136, 28, (0, 2)),
    (2, 512, 802816, 32, (0, 2)),
    (2, 128, 65536, 36, (1, 2)),
    (2, 256, 81920, 40, (1, 2)),
    (2, 512, 1064960, 44, (1, 2)),
]
PB = 320
NBLK = PP // PB

_CP = pltpu.CompilerParams(needs_layout_passes=False, use_tc_tiling_on_sc=False)


def _iota():
    return lax.iota(I32, 16)


def _cells(coord, R):
    pos = jnp.clip(coord, 0.0, 1.0) * (R - 1)
    cell = jnp.minimum(pos.astype(I32), R - 2)
    frac = pos - cell.astype(F32)
    return cell, frac


def _k1_body(pts, fts, p_out, chunk, xb, fb, zbuf, stage, idxb, fsem, zsem):
    cid = lax.axis_index("c")
    sid = lax.axis_index("s")
    wid = sid * 2 + cid
    iota = _iota()
    gbase = wid * PP

    for d in range(3):
        pltpu.sync_copy(pts.at[d, pl.ds(gbase, PP)], xb.at[d])
    for d in range(4):
        pltpu.sync_copy(fts.at[d, pl.ds(gbase, PP)], fb.at[d])

    def _z(r, carry):
        plsc.store_scatter(zbuf, [r * 2 + (iota >> 3), iota & 7],
                           jnp.zeros((16,), F32))
        return carry
    lax.fori_loop(0, 64, _z, 0)

    def drain_fj():
        for j in range(FJ):
            pltpu.make_async_copy(stage.at[0, pl.ds(j * 128, 128), :],
                                  chunk.at[idxb.at[0, j]], fsem).wait()

    for levels, nch, csz, pbase in JOBS:
        zq = csz // 16
        cs16 = csz // 16

        def one_chunk(c, levels=levels, nch=nch, csz=csz, pbase=pbase,
                      zq=zq, cs16=cs16):
            for k in range(zq // 128):
                pltpu.async_copy(zbuf, chunk.at[pl.ds(sid * zq + k * 128, 128), :], zsem)
            for k in range(zq // 128):
                pltpu.make_async_copy(zbuf, chunk.at[pl.ds(sid * zq + k * 128, 128), :], zsem).wait()
            plsc.subcore_barrier()

            cb = c * csz

            def vec(v, carry):
                ptr, b, fc = carry
                base = v * 16
                valid = (gbase + base + iota) < N
                fv = [fb[d, pl.ds(base, 16)] for d in range(4)]
                ib = jnp.full((16,), b, I32)
                for dims, R, loff, dd in levels:
                    pl_major = csz // (R * R) if dims == 3 else csz // R
                    cs_ = []
                    fr_ = []
                    for d in dd:
                        cell, frac = _cells(xb[d, pl.ds(base, 16)], R)
                        cs_.append(cell)
                        fr_.append(frac)
                    for corner in itertools.product((0, 1), repeat=dims):
                        w = jnp.ones((16,), F32)
                        for d in range(dims):
                            w = w * (fr_[d] if corner[d] == 1 else 1.0 - fr_[d])
                        cc0 = cs_[0] + corner[0]
                        flat = cc0
                        for d in range(1, dims):
                            flat = flat * R + cs_[d] + corner[d]
                        m = valid
                        if nch > 1:
                            lo = c * pl_major
                            m = m & (cc0 >= lo) & (cc0 < lo + pl_major)
                        lrow = flat - cb + loff
                        csum = plsc.cumsum(m.astype(I32))
                        pc = jnp.max(csum)
                        posv = ptr + csum - 1
                        vals = (w * fv[0], w * fv[1], w * fv[2], w * fv[3], w)
                        for ci, val in enumerate(vals):
                            plsc.store_scatter(stage, [ib, posv, jnp.full((16,), ci, I32)],
                                               val, mask=m)
                        plsc.store_scatter(idxb, [ib, posv >> 7, posv & 127], lrow, mask=m)
                        ptr = ptr + pc
                    ptr, b, fc = lax.cond(ptr >= CAP - 128, fire_fn, lambda a: a,
                                          (ptr, b, fc))
                return ptr, b, fc

            def fire_fn(a):
                ptr, b, fc = a
                ibf = jnp.full((16,), b, I32)
                for jj in range(8):
                    p = ptr + jj * 16 + iota
                    plsc.store_scatter(idxb, [ibf, p >> 7, p & 127],
                                       csz + (p & 127), mask=p < CAP)
                for j in range(FJ):
                    pltpu.async_copy(stage.at[b, pl.ds(j * 128, 128), :],
                                     chunk.at[idxb.at[b, j]], fsem, add=True)

                @pl.when(fc >= 1)
                def _():
                    drain_fj()
                return jnp.int32(0), 1 - b, fc + 1

            ptr, b, fc = lax.fori_loop(
                0, NV, vec, (jnp.int32(0), jnp.int32(0), jnp.int32(0)))

            ibf = jnp.full((16,), b, I32)
            for j in range(FJ):
                for jj in range(8):
                    p = j * 128 + jj * 16 + iota
                    plsc.store_scatter(idxb, [ibf, jnp.full((16,), j, I32),
                                              jj * 16 + iota],
                                       csz + (p & 127), mask=p >= ptr)

            for j in range(FJ):
                pltpu.async_copy(stage.at[b, pl.ds(j * 128, 128), :],
                                 chunk.at[idxb.at[b, j]], fsem, add=True)

            @pl.when(fc >= 1)
            def _():
                drain_fj()
            drain_fj()

            plsc.subcore_barrier()
            pltpu.sync_copy(chunk.at[pl.ds(sid * cs16, cs16), :],
                            p_out.at[cid, pl.ds(pbase + cb + sid * cs16, cs16), :])
            plsc.subcore_barrier()
            return 0

        if nch == 1:
            one_chunk(0)
        else:
            lax.fori_loop(0, nch, lambda c, _, f=one_chunk: (f(c), 0)[1], 0)


def _k2_body(p_in, g_out, pa, pb, og, rsem):
    cid = lax.axis_index("c")
    sid = lax.axis_index("s")
    wid = sid * 2 + cid
    iota = _iota()
    rt = TOT // 32
    rb_base = wid * rt

    def do_block(off, bn):
        h0 = pltpu.async_copy(p_in.at[0, pl.ds(off, bn), :], pa.at[pl.ds(0, bn), :], rsem)
        h1 = pltpu.async_copy(p_in.at[1, pl.ds(off, bn), :], pb.at[pl.ds(0, bn), :], rsem)
        h0.wait()
        h1.wait()

        def vec(vi, carry):
            r = vi * 16 + iota
            a = [plsc.load_gather(pa, [r, jnp.full((16,), ci, I32)]) for ci in range(5)]
            b = [plsc.load_gather(pb, [r, jnp.full((16,), ci, I32)]) for ci in range(5)]
            den = jnp.maximum(a[4] + b[4], 1e-8)
            for ci in range(4):
                plsc.store_scatter(og, [r, jnp.full((16,), ci, I32)],
                                   (a[ci] + b[ci]) / den)
            return carry
        lax.fori_loop(0, bn // 16, vec, 0)
        pltpu.sync_copy(og.at[pl.ds(0, bn), :], g_out.at[pl.ds(off, bn), :])

    def blk(i, carry):
        do_block(rb_base + i * 4096, 4096)
        return carry
    lax.fori_loop(0, rt // 4096, blk, 0)
    if rt % 4096:
        do_block(rb_base + (rt // 4096) * 4096, rt % 4096)


def _k3_body(qts, g_in, out, xq, wb, ibuf, rbuf, ob, gsem):
    cid = lax.axis_index("c")
    sid = lax.axis_index("s")
    wid = sid * 2 + cid
    iota = _iota()
    gbase = wid * PP
    for d in range(3):
        pltpu.sync_copy(qts.at[d, pl.ds(gbase, PP)], xq.at[d])

    segs = [(0, 128), (128, 128), (256, 64)]

    def blk(bi, carry):
        pbase = bi * PB
        handles = {}

        def prep(li):
            dims, R, goff, coff, dd = LEVELS[li]
            q = li % 2
            corners = list(itertools.product((0, 1), repeat=dims))

            def vec(v, c2):
                base = pbase + v * 16
                cs_, fr_ = [], []
                for d in dd:
                    cell, frac = _cells(xq[d, pl.ds(base, 16)], R)
                    cs_.append(cell)
                    fr_.append(frac)
                for ci, corner in enumerate(corners):
                    w = jnp.ones((16,), F32)
                    for d in range(dims):
                        w = w * (fr_[d] if corner[d] == 1 else 1.0 - fr_[d])
                    flat = cs_[0] + corner[0]
                    for d in range(1, dims):
                        flat = flat * R + cs_[d] + corner[d]
                    wb[q, ci, pl.ds(v * 16, 16)] = w
                    ibuf[q, ci, pl.ds(v * 16, 16)] = flat + goff
                return c2
            lax.fori_loop(0, PB // 16, vec, 0)
            hs = []
            for ci in range(len(corners)):
                for (o, ln) in segs:
                    hs.append(pltpu.async_copy(
                        g_in.at[ibuf.at[q, ci, pl.ds(o, ln)]],
                        rbuf.at[q, ci, pl.ds(o, ln), :], gsem.at[q]))
            handles[li] = hs

        def accum(li):
            dims, R, goff, coff, dd = LEVELS[li]
            q = li % 2
            ncor = 2 ** dims
            for h in handles[li]:
                h.wait()

            def grp(u, c2):
                r4 = u * 4 + (iota >> 2)
                colv = coff + (iota & 3)
                for ci in range(ncor):
                    rv = plsc.load_gather(rbuf.at[q, ci], [r4, iota & 3])
                    wv = plsc.load_gather(wb.at[q, ci], [r4])
                    val = rv * wv
                    if ci == 0:
                        plsc.store_scatter(ob, [r4, colv], val)
                    else:
                        plsc.addupdate_scatter(ob, [r4, colv], val)
                return c2
            lax.fori_loop(0, PB // 4, grp, 0)

        prep(0)
        for li in range(len(LEVELS)):
            if li + 1 < len(LEVELS):
                prep(li + 1)
            accum(li)
        pltpu.sync_copy(ob, out.at[pl.ds(gbase + pbase, PB), :])
        return carry

    lax.fori_loop(0, NBLK, blk, 0)


def kernel(xyz_for_creater, xyz_for_interp, feature):
    mesh = plsc.VectorSubcoreMesh(core_axis_name="c", subcore_axis_name="s")

    pts = jnp.pad(xyz_for_creater, ((0, NPAD - N), (0, 0))).T
    fts = jnp.pad(feature, ((0, NPAD - N), (0, 0))).T
    qts = jnp.pad(xyz_for_interp, ((0, NPAD - N), (0, 0))).T

    k1 = pl.kernel(
        _k1_body,
        out_type=jax.ShapeDtypeStruct((2, TOTP, 8), F32),
        mesh=mesh,
        compiler_params=_CP,
        scratch_types=[
            pltpu.VMEM_SHARED((CHS, 8), F32),
            pltpu.VMEM((3, PP), F32),
            pltpu.VMEM((4, PP), F32),
            pltpu.VMEM((128, 8), F32),
            pltpu.VMEM((2, CAP, 8), F32),
            pltpu.VMEM((2, FJ, 128), I32),
            pltpu.SemaphoreType.DMA,
            pltpu.SemaphoreType.DMA,
        ],
    )
    p_part = k1(pts, fts)

    k2 = pl.kernel(
        _k2_body,
        out_type=jax.ShapeDtypeStruct((TOT, 8), F32),
        mesh=mesh,
        compiler_params=_CP,
        scratch_types=[
            pltpu.VMEM((4096, 8), F32),
            pltpu.VMEM((4096, 8), F32),
            pltpu.VMEM((4096, 8), F32),
            pltpu.SemaphoreType.DMA,
        ],
    )
    g = k2(p_part)

    k3 = pl.kernel(
        _k3_body,
        out_type=jax.ShapeDtypeStruct((NPAD, 48), F32),
        mesh=mesh,
        compiler_params=_CP,
        scratch_types=[
            pltpu.VMEM((3, PP), F32),
            pltpu.VMEM((2, 8, PB), F32),
            pltpu.VMEM((2, 8, PB), I32),
            pltpu.VMEM((2, 8, PB, 8), F32),
            pltpu.VMEM((PB, 48), F32),
            pltpu.SemaphoreType.DMA((2,)),
        ],
    )
    out = k3(qts, g)
    return out[:N]
